# baseline (speedup 1.0000x reference)
import jax
import jax.numpy as jnp
from jax.experimental import pallas as pl
from jax.experimental.pallas import tpu as pltpu
from jax.experimental.pallas import tpu_sc as plsc

_L = 16
_NB = 51
_PAD = 64


def _lower_bound(chunks, x):
    total = None
    for v in chunks:
        cnt = plsc.all_reduce_population_count(v < x)
        total = cnt if total is None else total + cnt
    return total


def _body(pred_hbm, bins_hbm, theta_hbm, out_hbm, pred_v, bins_v, theta_v, out_v, sem):
    c1 = pltpu.async_copy(pred_hbm, pred_v.at[pl.ds(0, 1)], sem)
    c2 = pltpu.async_copy(bins_hbm, bins_v.at[pl.ds(0, _NB)], sem)
    c3 = pltpu.async_copy(theta_hbm, theta_v.at[pl.ds(0, _NB + 1)], sem)
    c1.wait()
    c2.wait()

    tail = bins_v[pl.ds(48, _L)]
    tail = jnp.where(jax.lax.iota(jnp.int32, _L) < 3, tail, jnp.inf)
    chunks = [bins_v[pl.ds(0, _L)], bins_v[pl.ds(_L, _L)], bins_v[pl.ds(2 * _L, _L)], tail]

    p = jnp.broadcast_to(pred_v[...][0], (_L,))
    j = _lower_bound(chunks, p)
    c3.wait()
    binned = plsc.load_gather(bins_v, [jnp.minimum(j, _NB - 1)])
    adj = binned + plsc.load_gather(theta_v, [j])
    i = _lower_bound(chunks, adj)
    out_v[...] = plsc.load_gather(bins_v, [jnp.minimum(i, _NB - 1)])
    pltpu.sync_copy(out_v.at[pl.ds(0, 1)], out_hbm)


def kernel(prediction, bin_values, theta):
    f = pl.kernel(
        _body,
        mesh=plsc.VectorSubcoreMesh(
            core_axis_name="c", subcore_axis_name="s", num_cores=1, num_subcores=1
        ),
        out_type=jax.ShapeDtypeStruct((1,), jnp.float32),
        scratch_types=[
            pltpu.VMEM((_L,), jnp.float32),
            pltpu.VMEM((_PAD,), jnp.float32),
            pltpu.VMEM((_PAD,), jnp.float32),
            pltpu.VMEM((_L,), jnp.float32),
            pltpu.SemaphoreType.DMA,
        ],
        compiler_params=pltpu.CompilerParams(needs_layout_passes=False),
    )
    out = f(jnp.reshape(prediction, (1,)), bin_values, theta)
    return jnp.reshape(out, ())
